# baseline (speedup 1.0000x reference)
import functools

import jax
import jax.numpy as jnp
from jax import lax
from jax.experimental import pallas as pl
from jax.experimental.pallas import tpu as pltpu, tpu_sc as plsc

_EMB = 1000
_BATCH = 16384
_NC = 2
_NS = 16
_NW = _NC * _NS
_ROWS_PER_W = _BATCH // _NW
_CHUNK = 16
_NCHUNK = _ROWS_PER_W // _CHUNK
_NBUF = 2

_mesh = plsc.VectorSubcoreMesh(core_axis_name="c", subcore_axis_name="s")


@functools.partial(
    pl.kernel,
    out_type=jax.ShapeDtypeStruct((_BATCH, _EMB), jnp.float32),
    mesh=_mesh,
    compiler_params=pltpu.CompilerParams(
        needs_layout_passes=False, use_tc_tiling_on_sc=True
    ),
    scratch_types=[
        pltpu.VMEM((_ROWS_PER_W,), jnp.int32),
        pltpu.VMEM((_NBUF * _CHUNK, _EMB), jnp.float32),
        pltpu.SemaphoreType.DMA,
        pltpu.SemaphoreType.DMA,
    ],
)
def _one_hot_sc(labels_hbm, out_hbm, lab_v, buf_v, sem0, sem1):
    wid = lax.axis_index("s") * _NC + lax.axis_index("c")
    row0 = wid * _ROWS_PER_W

    pltpu.sync_copy(labels_hbm.at[pl.ds(row0, _ROWS_PER_W)], lab_v)

    zeros = jnp.zeros((16,), jnp.float32)
    ones = jnp.ones((16,), jnp.float32)
    iota16 = lax.broadcasted_iota(jnp.int32, (16,), 0)

    def _zero_row(r, _):
        for j in range(_EMB // 16):
            buf_v[r, pl.ds(j * 16, 16)] = zeros
        buf_v[r, pl.ds(_EMB - 16, 16)] = zeros
        return _

    lax.fori_loop(0, _NBUF * _CHUNK, _zero_row, None)

    sems = (sem0, sem1)

    def _scatter(c, half, vals):
        lab16 = lab_v[pl.ds(c * _CHUNK, _CHUNK)]
        plsc.store_scatter(buf_v, [half * _CHUNK + iota16, lab16], vals)

    pending = [None] * _NBUF
    for c in range(_NCHUNK):
        half = c % _NBUF
        if pending[half] is not None:
            cp, old_c = pending[half]
            cp.wait()
            _scatter(old_c, half, zeros)
        _scatter(c, half, ones)
        cp = pltpu.async_copy(
            buf_v.at[pl.ds(half * _CHUNK, _CHUNK)],
            out_hbm.at[pl.ds(row0 + c * _CHUNK, _CHUNK)],
            sems[half],
        )
        pending[half] = (cp, c)

    for half in range(_NBUF):
        if pending[half] is not None:
            pending[half][0].wait()


def kernel(labels):
    return _one_hot_sc(labels)


# speedup vs baseline: 2.2266x; 2.2266x over previous
import functools

import jax
import jax.numpy as jnp
from jax import lax
from jax.experimental import pallas as pl
from jax.experimental.pallas import tpu as pltpu, tpu_sc as plsc

_EMB = 1000
_BATCH = 16384
_NC = 2
_NS = 16
_NW = _NC * _NS
_COLS_PER_W = _BATCH // _NW
_CHUNK = 128
_NCHUNK = _COLS_PER_W // _CHUNK

_mesh = plsc.VectorSubcoreMesh(core_axis_name="c", subcore_axis_name="s")


@functools.partial(
    pl.kernel,
    out_type=jax.ShapeDtypeStruct((_EMB, _BATCH), jnp.float32),
    mesh=_mesh,
    compiler_params=pltpu.CompilerParams(needs_layout_passes=False),
    scratch_types=[
        pltpu.VMEM((_COLS_PER_W,), jnp.int32),
        pltpu.VMEM((_EMB, _CHUNK), jnp.float32),
        pltpu.SemaphoreType.DMA,
    ],
)
def _one_hot_sc(labels_hbm, out_hbm, lab_v, buf_v, sem):
    wid = lax.axis_index("s") * _NC + lax.axis_index("c")
    col0 = wid * _COLS_PER_W

    pltpu.sync_copy(labels_hbm.at[pl.ds(col0, _COLS_PER_W)], lab_v)

    zeros = jnp.zeros((16,), jnp.float32)
    ones = jnp.ones((16,), jnp.float32)
    iota16 = lax.broadcasted_iota(jnp.int32, (16,), 0)

    def _zero_row(r, _):
        for j in range(_CHUNK // 16):
            buf_v[r, pl.ds(j * 16, 16)] = zeros
        return _

    lax.fori_loop(0, _EMB, _zero_row, None)

    def _scatter(c, vals):
        for g in range(_CHUNK // 16):
            lab16 = lab_v[pl.ds(c * _CHUNK + g * 16, 16)]
            plsc.store_scatter(buf_v, [lab16, g * 16 + iota16], vals)

    for c in range(_NCHUNK):
        _scatter(c, ones)
        pltpu.async_copy(
            buf_v,
            out_hbm.at[:, pl.ds(col0 + c * _CHUNK, _CHUNK)],
            sem,
        ).wait()
        if c + 1 < _NCHUNK:
            _scatter(c, zeros)


def kernel(labels):
    return _one_hot_sc(labels).T
